# baseline (speedup 1.0000x reference)
import functools

import jax
import jax.numpy as jnp
from jax import lax
from jax.experimental import pallas as pl
from jax.experimental.pallas import tpu as pltpu
from jax.experimental.pallas import tpu_sc as plsc

_NL = 7
_TILE = 2048


def _sc_gather(latents, idx):
    B = idx.shape[0]
    L = latents.shape[1]
    mesh = plsc.VectorSubcoreMesh(core_axis_name="c", subcore_axis_name="s")

    @functools.partial(
        pl.kernel,
        out_type=jax.ShapeDtypeStruct((B, L), jnp.float32),
        mesh=mesh,
        scratch_types=[
            pltpu.VMEM((B,), jnp.int32),
            pltpu.VMEM((B, L), jnp.float32),
            pltpu.SemaphoreType.DMA,
        ],
    )
    def gather_kernel(table_hbm, idx_hbm, out_hbm, idx_v, rows_v, sem):
        first = (lax.axis_index("c") == 0) & (lax.axis_index("s") == 0)

        @pl.when(first)
        def _():
            pltpu.sync_copy(idx_hbm, idx_v)
            pltpu.async_copy(table_hbm.at[idx_v], rows_v, sem).wait()
            pltpu.sync_copy(rows_v, out_hbm)

    return gather_kernel(latents, idx)


def _trunk_body(x_ref, lat_ref, At_ref, Bt_ref, bias_ref, gWt_ref, gb_ref,
                mut_ref, gam_ref, WoutT_ref, bout_ref, out_ref):
    bf = jnp.bfloat16
    f32 = jnp.float32

    def rnd(v):
        return v.astype(bf).astype(f32)

    lat = lat_ref[0].astype(bf)
    ll = [
        lax.dot_general(lat, Bt_ref[i], (((1,), (0,)), ((), ())),
                        preferred_element_type=f32) + bias_ref[i]
        for i in range(_NL + 1)
    ]

    x0 = x_ref[0, :, 0:1]
    x1 = x_ref[0, :, 1:2]
    xsq = x0 * x0 + x1 * x1
    x0b = rnd(x0)
    x1b = rnd(x1)

    def gabor(i):
        w0 = rnd(gWt_ref[i, 0:1, :])
        w1 = rnd(gWt_ref[i, 1:2, :])
        m0 = mut_ref[i, 0:1, :]
        m1 = mut_ref[i, 1:2, :]
        musq = m0 * m0 + m1 * m1
        xm = x0b * rnd(m0) + x1b * rnd(m1)
        d = (xsq + musq - 2.0 * xm) * gam_ref[i]
        return jnp.sin(x0b * w0 + x1b * w1 + gb_ref[i]) * jnp.exp(-0.5 * d)

    z = gabor(0) * ll[0]
    for i in range(1, _NL + 1):
        zi = lax.dot_general(z.astype(bf), At_ref[i - 1],
                             (((1,), (0,)), ((), ())),
                             preferred_element_type=f32)
        z = (zi + ll[i]) * gabor(i)

    res = lax.dot_general(z.astype(bf), WoutT_ref[...],
                          (((1,), (0,)), ((), ())),
                          preferred_element_type=f32) + bout_ref[...]
    out_ref[...] = res[None]


def kernel(x, idx, latents, A0, A_rest, B_all, bias_all, gW, gb, mu, gamma,
           Wout, bout):
    del A0
    B, P, _ = x.shape
    H = A_rest.shape[1]
    L = B_all.shape[2]
    T = _TILE
    PT = P // T

    latent = _sc_gather(latents, idx)[:, None, :]

    At = A_rest.transpose(0, 2, 1).astype(jnp.bfloat16)
    Bt = B_all.transpose(0, 2, 1).astype(jnp.bfloat16)
    gWt = gW.transpose(0, 2, 1)
    mut = mu.transpose(0, 2, 1)
    bias3 = bias_all[:, None, :]
    gb3 = gb[:, None, :]
    gam3 = gamma[:, None, :]
    WoutT = Wout.T.astype(jnp.bfloat16)
    bout2 = bout[None, :]

    full = lambda s: pl.BlockSpec(s, lambda b, p: (0,) * len(s))
    out = pl.pallas_call(
        _trunk_body,
        grid=(B, PT),
        in_specs=[
            pl.BlockSpec((1, T, 2), lambda b, p: (b, p, 0)),
            pl.BlockSpec((1, 1, L), lambda b, p: (b, 0, 0)),
            full((_NL, H, H)),
            full((_NL + 1, L, H)),
            full((_NL + 1, 1, H)),
            full((_NL + 1, 2, H)),
            full((_NL + 1, 1, H)),
            full((_NL + 1, 2, H)),
            full((_NL + 1, 1, H)),
            full((H, 1)),
            full((1, 1)),
        ],
        out_specs=pl.BlockSpec((1, T, 1), lambda b, p: (b, p, 0)),
        out_shape=jax.ShapeDtypeStruct((B, P, 1), jnp.float32),
        compiler_params=pltpu.CompilerParams(
            dimension_semantics=("parallel", "parallel"),
        ),
    )(x, latent, At, Bt, bias3, gWt, gb3, mut, gam3, WoutT, bout2)
    return out


# speedup vs baseline: 3.2492x; 3.2492x over previous
import functools

import jax
import jax.numpy as jnp
from jax import lax
from jax.experimental import pallas as pl
from jax.experimental.pallas import tpu as pltpu
from jax.experimental.pallas import tpu_sc as plsc

_NL = 7
_TILE = 2048

_INV_PI = 0.3183098861837907
_PI1 = 3.140625
_PI2 = 0.0009675025939941406
_PI3 = 1.509957990783135e-07
_S1 = -1.6666654611e-01
_S2 = 8.3321608736e-03
_S3 = -1.9515295891e-04


def _fast_sin(s):
    n = jnp.floor(s * _INV_PI + 0.5)
    r = (s - n * _PI1) - n * _PI2
    r = r - n * _PI3
    odd = lax.shift_left(jnp.bitwise_and(n.astype(jnp.int32), 1), 31)
    r = lax.bitcast_convert_type(
        lax.bitcast_convert_type(r, jnp.int32) ^ odd, jnp.float32)
    r2 = r * r
    return r + r * r2 * (_S1 + r2 * (_S2 + r2 * _S3))


def _sc_gather(latents, idx):
    B = idx.shape[0]
    L = latents.shape[1]
    mesh = plsc.VectorSubcoreMesh(core_axis_name="c", subcore_axis_name="s")

    @functools.partial(
        pl.kernel,
        out_type=jax.ShapeDtypeStruct((B, L), jnp.float32),
        mesh=mesh,
        scratch_types=[
            pltpu.VMEM((B,), jnp.int32),
            pltpu.VMEM((B, L), jnp.float32),
            pltpu.SemaphoreType.DMA,
        ],
    )
    def gather_kernel(table_hbm, idx_hbm, out_hbm, idx_v, rows_v, sem):
        first = (lax.axis_index("c") == 0) & (lax.axis_index("s") == 0)

        @pl.when(first)
        def _():
            pltpu.sync_copy(idx_hbm, idx_v)
            pltpu.async_copy(table_hbm.at[idx_v], rows_v, sem).wait()
            pltpu.sync_copy(rows_v, out_hbm)

    return gather_kernel(latents, idx)


def _trunk_body(x_ref, lat_ref, At_ref, Bt_ref, bias_ref, gWt_ref, gb_ref,
                mut_ref, gam_ref, WoutT_ref, bout_ref, out_ref):
    bf = jnp.bfloat16
    f32 = jnp.float32

    def rnd(v):
        return v.astype(bf).astype(f32)

    lat = lat_ref[0].astype(bf)
    ll = [
        lax.dot_general(lat, Bt_ref[i], (((1,), (0,)), ((), ())),
                        preferred_element_type=f32) + bias_ref[i]
        for i in range(_NL + 1)
    ]

    x0 = x_ref[0, :, 0:1]
    x1 = x_ref[0, :, 1:2]
    xsq = x0 * x0 + x1 * x1
    x0b = rnd(x0)
    x1b = rnd(x1)

    def gabor(i):
        w0 = rnd(gWt_ref[i, 0:1, :])
        w1 = rnd(gWt_ref[i, 1:2, :])
        m0 = mut_ref[i, 0:1, :]
        m1 = mut_ref[i, 1:2, :]
        musq = m0 * m0 + m1 * m1
        xm = x0b * rnd(m0) + x1b * rnd(m1)
        d = (xsq + musq - 2.0 * xm) * gam_ref[i]
        return _fast_sin(x0b * w0 + x1b * w1 + gb_ref[i]) * jnp.exp(-0.5 * d)

    z = gabor(0) * ll[0]
    for i in range(1, _NL + 1):
        zi = lax.dot_general(z.astype(bf), At_ref[i - 1],
                             (((1,), (0,)), ((), ())),
                             preferred_element_type=f32)
        z = (zi + ll[i]) * gabor(i)

    res = lax.dot_general(z.astype(bf), WoutT_ref[...],
                          (((1,), (0,)), ((), ())),
                          preferred_element_type=f32) + bout_ref[...]
    out_ref[...] = res[None]


def kernel(x, idx, latents, A0, A_rest, B_all, bias_all, gW, gb, mu, gamma,
           Wout, bout):
    del A0
    B, P, _ = x.shape
    H = A_rest.shape[1]
    L = B_all.shape[2]
    T = _TILE
    PT = P // T

    latent = _sc_gather(latents, idx)[:, None, :]

    At = A_rest.transpose(0, 2, 1).astype(jnp.bfloat16)
    Bt = B_all.transpose(0, 2, 1).astype(jnp.bfloat16)
    gWt = gW.transpose(0, 2, 1)
    mut = mu.transpose(0, 2, 1)
    bias3 = bias_all[:, None, :]
    gb3 = gb[:, None, :]
    gam3 = gamma[:, None, :]
    WoutT = Wout.T.astype(jnp.bfloat16)
    bout2 = bout[None, :]

    full = lambda s: pl.BlockSpec(s, lambda b, p: (0,) * len(s))
    out = pl.pallas_call(
        _trunk_body,
        grid=(B, PT),
        in_specs=[
            pl.BlockSpec((1, T, 2), lambda b, p: (b, p, 0)),
            pl.BlockSpec((1, 1, L), lambda b, p: (b, 0, 0)),
            full((_NL, H, H)),
            full((_NL + 1, L, H)),
            full((_NL + 1, 1, H)),
            full((_NL + 1, 2, H)),
            full((_NL + 1, 1, H)),
            full((_NL + 1, 2, H)),
            full((_NL + 1, 1, H)),
            full((H, 1)),
            full((1, 1)),
        ],
        out_specs=pl.BlockSpec((1, T, 1), lambda b, p: (b, p, 0)),
        out_shape=jax.ShapeDtypeStruct((B, P, 1), jnp.float32),
        compiler_params=pltpu.CompilerParams(
            dimension_semantics=("parallel", "parallel"),
        ),
    )(x, latent, At, Bt, bias3, gWt, gb3, mut, gam3, WoutT, bout2)
    return out


# speedup vs baseline: 4.1294x; 1.2709x over previous
import functools

import jax
import jax.numpy as jnp
from jax import lax
from jax.experimental import pallas as pl
from jax.experimental.pallas import tpu as pltpu
from jax.experimental.pallas import tpu_sc as plsc

_NL = 7
_TILE = 2048

_INV_PI = 0.3183098861837907
_PI1 = 3.140625
_PI2 = 0.0009675025939941406
_PI3 = 1.509957990783135e-07
_S1 = -1.6666654611e-01
_S2 = 8.3321608736e-03
_S3 = -1.9515295891e-04


def _fast_sin(s):
    n = jnp.floor(s * _INV_PI + 0.5)
    r = (s - n * _PI1) - n * _PI2
    odd = lax.shift_left(jnp.bitwise_and(n.astype(jnp.int32), 1), 31)
    r = lax.bitcast_convert_type(
        lax.bitcast_convert_type(r, jnp.int32) ^ odd, jnp.float32)
    r2 = r * r
    return r + r * r2 * (_S1 + r2 * (_S2 + r2 * _S3))


def _sc_gather(latents, idx):
    B = idx.shape[0]
    L = latents.shape[1]
    mesh = plsc.VectorSubcoreMesh(core_axis_name="c", subcore_axis_name="s")

    @functools.partial(
        pl.kernel,
        out_type=jax.ShapeDtypeStruct((B, L), jnp.float32),
        mesh=mesh,
        scratch_types=[
            pltpu.VMEM((B,), jnp.int32),
            pltpu.VMEM((B, L), jnp.float32),
            pltpu.SemaphoreType.DMA,
        ],
    )
    def gather_kernel(table_hbm, idx_hbm, out_hbm, idx_v, rows_v, sem):
        first = (lax.axis_index("c") == 0) & (lax.axis_index("s") == 0)

        @pl.when(first)
        def _():
            pltpu.sync_copy(idx_hbm, idx_v)
            pltpu.async_copy(table_hbm.at[idx_v], rows_v, sem).wait()
            pltpu.sync_copy(rows_v, out_hbm)

    return gather_kernel(latents, idx)


def _trunk_body(x_ref, lat_ref, At_ref, Bt_ref, bias_ref, Wcat_ref, gb_ref,
                mut_ref, Mcat_ref, gamn_ref, WoutT_ref, bout_ref, out_ref):
    bf = jnp.bfloat16
    f32 = jnp.float32
    nn = (((1,), (0,)), ((), ()))

    lat = lat_ref[0].astype(bf)
    ll = [
        lax.dot_general(lat, Bt_ref[i], nn, preferred_element_type=f32)
        + bias_ref[i]
        for i in range(_NL + 1)
    ]

    x0 = x_ref[0, :, 0:1]
    x1 = x_ref[0, :, 1:2]
    xsq = x0 * x0 + x1 * x1
    xb = x_ref[0].astype(bf)

    def gabor(i):
        m0 = mut_ref[i, 0:1, :]
        m1 = mut_ref[i, 1:2, :]
        musq = m0 * m0 + m1 * m1
        xm2 = lax.dot_general(xb, Mcat_ref[i], nn, preferred_element_type=f32)
        sarg = lax.dot_general(xb, Wcat_ref[i], nn, preferred_element_type=f32)
        e = jnp.exp2(((xsq + musq) - xm2) * gamn_ref[i])
        return _fast_sin(sarg + gb_ref[i]) * e

    z = gabor(0) * ll[0]
    for i in range(1, _NL + 1):
        zi = lax.dot_general(z.astype(bf), At_ref[i - 1],
                             (((1,), (0,)), ((), ())),
                             preferred_element_type=f32)
        z = (zi + ll[i]) * gabor(i)

    res = lax.dot_general(z.astype(bf), WoutT_ref[...],
                          (((1,), (0,)), ((), ())),
                          preferred_element_type=f32) + bout_ref[...]
    out_ref[...] = res[None]


def kernel(x, idx, latents, A0, A_rest, B_all, bias_all, gW, gb, mu, gamma,
           Wout, bout):
    del A0
    B, P, _ = x.shape
    H = A_rest.shape[1]
    L = B_all.shape[2]
    T = _TILE
    PT = P // T

    latent = _sc_gather(latents, idx)[:, None, :]

    At = A_rest.transpose(0, 2, 1).astype(jnp.bfloat16)
    Bt = B_all.transpose(0, 2, 1).astype(jnp.bfloat16)
    Wcat = gW.transpose(0, 2, 1).astype(jnp.bfloat16)
    mut = mu.transpose(0, 2, 1)
    Mcat = mut.astype(jnp.bfloat16) * jnp.bfloat16(2.0)
    bias3 = bias_all[:, None, :]
    gb3 = gb[:, None, :]
    gamn = gamma[:, None, :] * jnp.float32(-0.5 * 1.4426950408889634)
    WoutT = Wout.T.astype(jnp.bfloat16)
    bout2 = bout[None, :]

    full = lambda s: pl.BlockSpec(s, lambda b, p: (0,) * len(s))
    out = pl.pallas_call(
        _trunk_body,
        grid=(B, PT),
        in_specs=[
            pl.BlockSpec((1, T, 2), lambda b, p: (b, p, 0)),
            pl.BlockSpec((1, 1, L), lambda b, p: (b, 0, 0)),
            full((_NL, H, H)),
            full((_NL + 1, L, H)),
            full((_NL + 1, 1, H)),
            full((_NL + 1, 2, H)),
            full((_NL + 1, 1, H)),
            full((_NL + 1, 2, H)),
            full((_NL + 1, 2, H)),
            full((_NL + 1, 1, H)),
            full((H, 1)),
            full((1, 1)),
        ],
        out_specs=pl.BlockSpec((1, T, 1), lambda b, p: (b, p, 0)),
        out_shape=jax.ShapeDtypeStruct((B, P, 1), jnp.float32),
        compiler_params=pltpu.CompilerParams(
            dimension_semantics=("parallel", "parallel"),
        ),
    )(x, latent, At, Bt, bias3, Wcat, gb3, mut, Mcat, gamn, WoutT, bout2)
    return out


# speedup vs baseline: 4.9319x; 1.1943x over previous
import functools

import jax
import jax.numpy as jnp
from jax import lax
from jax.experimental import pallas as pl
from jax.experimental.pallas import tpu as pltpu
from jax.experimental.pallas import tpu_sc as plsc

_NL = 7
_TILE = 2048

_INV_PI = 0.3183098861837907
_MAGIC = 12582912.0
_C1 = 3.14158198
_C3 = -5.1671413
_C5 = 2.54188707
_C7 = -0.55460885


def _fast_sin_phase(p):
    big = p + _MAGIC
    n = big - _MAGIC
    u = p - n
    odd = lax.shift_left(lax.bitcast_convert_type(big, jnp.int32), 31)
    u = lax.bitcast_convert_type(
        lax.bitcast_convert_type(u, jnp.int32) ^ odd, jnp.float32)
    u2 = u * u
    return u * (_C1 + u2 * (_C3 + u2 * (_C5 + u2 * _C7)))


def _sc_gather(latents, idx):
    B = idx.shape[0]
    L = latents.shape[1]
    mesh = plsc.VectorSubcoreMesh(core_axis_name="c", subcore_axis_name="s")

    @functools.partial(
        pl.kernel,
        out_type=jax.ShapeDtypeStruct((B, L), jnp.float32),
        mesh=mesh,
        scratch_types=[
            pltpu.VMEM((B,), jnp.int32),
            pltpu.VMEM((B, L), jnp.float32),
            pltpu.SemaphoreType.DMA,
        ],
    )
    def gather_kernel(table_hbm, idx_hbm, out_hbm, idx_v, rows_v, sem):
        first = (lax.axis_index("c") == 0) & (lax.axis_index("s") == 0)

        @pl.when(first)
        def _():
            pltpu.sync_copy(idx_hbm, idx_v)
            pltpu.async_copy(table_hbm.at[idx_v], rows_v, sem).wait()
            pltpu.sync_copy(rows_v, out_hbm)

    return gather_kernel(latents, idx)


def _trunk_body(x_ref, lat_ref, At_ref, Bt_ref, bias_ref, Wcat_ref, gb_ref,
                mut_ref, Mcat_ref, gamn_ref, WoutT_ref, bout_ref, out_ref):
    bf = jnp.bfloat16
    f32 = jnp.float32
    nn = (((1,), (0,)), ((), ()))

    lat = lat_ref[0].astype(bf)
    ll = [
        lax.dot_general(lat, Bt_ref[i], nn, preferred_element_type=f32)
        + bias_ref[i]
        for i in range(_NL + 1)
    ]

    x0 = x_ref[0, :, 0:1]
    x1 = x_ref[0, :, 1:2]
    xsq = x0 * x0 + x1 * x1
    xb = x_ref[0].astype(bf)

    def gabor(i):
        m0 = mut_ref[i, 0:1, :]
        m1 = mut_ref[i, 1:2, :]
        musq = m0 * m0 + m1 * m1
        xm2 = lax.dot_general(xb, Mcat_ref[i], nn, preferred_element_type=f32)
        sarg = lax.dot_general(xb, Wcat_ref[i], nn, preferred_element_type=f32)
        e = jnp.exp2(((xsq + musq) - xm2) * gamn_ref[i])
        return _fast_sin_phase(sarg * _INV_PI + gb_ref[i]) * e

    z = gabor(0) * ll[0]
    for i in range(1, _NL + 1):
        zi = lax.dot_general(z.astype(bf), At_ref[i - 1],
                             (((1,), (0,)), ((), ())),
                             preferred_element_type=f32)
        z = (zi + ll[i]) * gabor(i)

    res = lax.dot_general(z.astype(bf), WoutT_ref[...],
                          (((1,), (0,)), ((), ())),
                          preferred_element_type=f32) + bout_ref[...]
    out_ref[...] = res[None]


def kernel(x, idx, latents, A0, A_rest, B_all, bias_all, gW, gb, mu, gamma,
           Wout, bout):
    del A0
    B, P, _ = x.shape
    H = A_rest.shape[1]
    L = B_all.shape[2]
    T = _TILE
    PT = P // T

    latent = _sc_gather(latents, idx)[:, None, :]

    At = A_rest.transpose(0, 2, 1).astype(jnp.bfloat16)
    Bt = B_all.transpose(0, 2, 1).astype(jnp.bfloat16)
    Wcat = gW.transpose(0, 2, 1).astype(jnp.bfloat16)
    mut = mu.transpose(0, 2, 1)
    Mcat = mut.astype(jnp.bfloat16) * jnp.bfloat16(2.0)
    bias3 = bias_all[:, None, :]
    gb3 = gb[:, None, :] * jnp.float32(_INV_PI)
    gamn = gamma[:, None, :] * jnp.float32(-0.5 * 1.4426950408889634)
    WoutT = Wout.T.astype(jnp.bfloat16)
    bout2 = bout[None, :]

    full = lambda s: pl.BlockSpec(s, lambda b, p: (0,) * len(s))
    out = pl.pallas_call(
        _trunk_body,
        grid=(B, PT),
        in_specs=[
            pl.BlockSpec((1, T, 2), lambda b, p: (b, p, 0)),
            pl.BlockSpec((1, 1, L), lambda b, p: (b, 0, 0)),
            full((_NL, H, H)),
            full((_NL + 1, L, H)),
            full((_NL + 1, 1, H)),
            full((_NL + 1, 2, H)),
            full((_NL + 1, 1, H)),
            full((_NL + 1, 2, H)),
            full((_NL + 1, 2, H)),
            full((_NL + 1, 1, H)),
            full((H, 1)),
            full((1, 1)),
        ],
        out_specs=pl.BlockSpec((1, T, 1), lambda b, p: (b, p, 0)),
        out_shape=jax.ShapeDtypeStruct((B, P, 1), jnp.float32),
        compiler_params=pltpu.CompilerParams(
            dimension_semantics=("parallel", "parallel"),
        ),
    )(x, latent, At, Bt, bias3, Wcat, gb3, mut, Mcat, gamn, WoutT, bout2)
    return out


# speedup vs baseline: 5.5125x; 1.1177x over previous
import functools

import jax
import jax.numpy as jnp
from jax import lax
from jax.experimental import pallas as pl
from jax.experimental.pallas import tpu as pltpu
from jax.experimental.pallas import tpu_sc as plsc

_NL = 7
_TILE = 2048

_INV_PI = 0.3183098861837907
_MAGIC = 12582912.0
_C1 = 3.14063416
_C3 = -5.13681113
_C5 = 2.29924569


def _fast_sin_phase(p):
    big = p + _MAGIC
    n = big - _MAGIC
    u = p - n
    odd = lax.shift_left(lax.bitcast_convert_type(big, jnp.int32), 31)
    u = lax.bitcast_convert_type(
        lax.bitcast_convert_type(u, jnp.int32) ^ odd, jnp.float32)
    u2 = u * u
    return u * (_C1 + u2 * (_C3 + u2 * _C5))


def _sc_gather(latents, idx):
    B = idx.shape[0]
    L = latents.shape[1]
    mesh = plsc.VectorSubcoreMesh(core_axis_name="c", subcore_axis_name="s")

    @functools.partial(
        pl.kernel,
        out_type=jax.ShapeDtypeStruct((B, L), jnp.float32),
        mesh=mesh,
        scratch_types=[
            pltpu.VMEM((B,), jnp.int32),
            pltpu.VMEM((B, L), jnp.float32),
            pltpu.SemaphoreType.DMA,
        ],
    )
    def gather_kernel(table_hbm, idx_hbm, out_hbm, idx_v, rows_v, sem):
        first = (lax.axis_index("c") == 0) & (lax.axis_index("s") == 0)

        @pl.when(first)
        def _():
            pltpu.sync_copy(idx_hbm, idx_v)
            pltpu.async_copy(table_hbm.at[idx_v], rows_v, sem).wait()
            pltpu.sync_copy(rows_v, out_hbm)

    return gather_kernel(latents, idx)


def _trunk_body(x_ref, lat_ref, At_ref, Bt_ref, bias_ref, GM_ref, gb_ref,
                mut_ref, gamn_ref, WoutT_ref, bout_ref, out_ref):
    bf = jnp.bfloat16
    f32 = jnp.float32
    nn = (((1,), (0,)), ((), ()))
    H = At_ref.shape[1]

    lat = lat_ref[0].astype(bf)
    ll = [
        lax.dot_general(lat, Bt_ref[i], nn, preferred_element_type=f32)
        + bias_ref[i]
        for i in range(_NL + 1)
    ]

    x0 = x_ref[0, :, 0:1]
    x1 = x_ref[0, :, 1:2]
    xsq = x0 * x0 + x1 * x1
    xb = x_ref[0].astype(bf)

    def gabor(i):
        m0 = mut_ref[i, 0:1, :]
        m1 = mut_ref[i, 1:2, :]
        musq = m0 * m0 + m1 * m1
        sm = lax.dot_general(xb, GM_ref[i], nn, preferred_element_type=f32)
        e = jnp.exp2(((xsq + musq) - sm[:, H:]) * gamn_ref[i])
        return _fast_sin_phase(sm[:, :H] * _INV_PI + gb_ref[i]) * e

    z = gabor(0) * ll[0]
    for i in range(1, _NL + 1):
        zi = lax.dot_general(z.astype(bf), At_ref[i - 1],
                             (((1,), (0,)), ((), ())),
                             preferred_element_type=f32)
        z = (zi + ll[i]) * gabor(i)

    res = lax.dot_general(z.astype(bf), WoutT_ref[...],
                          (((1,), (0,)), ((), ())),
                          preferred_element_type=f32) + bout_ref[...]
    out_ref[...] = res[None]


def kernel(x, idx, latents, A0, A_rest, B_all, bias_all, gW, gb, mu, gamma,
           Wout, bout):
    del A0
    B, P, _ = x.shape
    H = A_rest.shape[1]
    L = B_all.shape[2]
    T = _TILE
    PT = P // T

    latent = _sc_gather(latents, idx)[:, None, :]

    At = A_rest.transpose(0, 2, 1).astype(jnp.bfloat16)
    Bt = B_all.transpose(0, 2, 1).astype(jnp.bfloat16)
    Wcat = gW.transpose(0, 2, 1).astype(jnp.bfloat16)
    mut = mu.transpose(0, 2, 1)
    Mcat = mut.astype(jnp.bfloat16) * jnp.bfloat16(2.0)
    GM = jnp.concatenate([Wcat, Mcat], axis=2)
    bias3 = bias_all[:, None, :]
    gb3 = gb[:, None, :] * jnp.float32(_INV_PI)
    gamn = gamma[:, None, :] * jnp.float32(-0.5 * 1.4426950408889634)
    WoutT = Wout.T.astype(jnp.bfloat16)
    bout2 = bout[None, :]

    full = lambda s: pl.BlockSpec(s, lambda b, p: (0,) * len(s))
    out = pl.pallas_call(
        _trunk_body,
        grid=(B, PT),
        in_specs=[
            pl.BlockSpec((1, T, 2), lambda b, p: (b, p, 0)),
            pl.BlockSpec((1, 1, L), lambda b, p: (b, 0, 0)),
            full((_NL, H, H)),
            full((_NL + 1, L, H)),
            full((_NL + 1, 1, H)),
            full((_NL + 1, 2, 2 * H)),
            full((_NL + 1, 1, H)),
            full((_NL + 1, 2, H)),
            full((_NL + 1, 1, H)),
            full((H, 1)),
            full((1, 1)),
        ],
        out_specs=pl.BlockSpec((1, T, 1), lambda b, p: (b, p, 0)),
        out_shape=jax.ShapeDtypeStruct((B, P, 1), jnp.float32),
        compiler_params=pltpu.CompilerParams(
            dimension_semantics=("parallel", "parallel"),
        ),
    )(x, latent, At, Bt, bias3, GM, gb3, mut, gamn, WoutT, bout2)
    return out


# speedup vs baseline: 5.6216x; 1.0198x over previous
import functools

import jax
import jax.numpy as jnp
from jax import lax
from jax.experimental import pallas as pl
from jax.experimental.pallas import tpu as pltpu
from jax.experimental.pallas import tpu_sc as plsc

_NL = 7
_TILE = 4096

_INV_PI = 0.3183098861837907
_MAGIC = 12582912.0
_C1 = 3.14063416
_C3 = -5.13681113
_C5 = 2.29924569


def _fast_sin_phase(p):
    big = p + _MAGIC
    n = big - _MAGIC
    u = p - n
    odd = lax.shift_left(lax.bitcast_convert_type(big, jnp.int32), 31)
    u = lax.bitcast_convert_type(
        lax.bitcast_convert_type(u, jnp.int32) ^ odd, jnp.float32)
    u2 = u * u
    return u * (_C1 + u2 * (_C3 + u2 * _C5))


def _sc_gather(latents, idx):
    B = idx.shape[0]
    L = latents.shape[1]
    mesh = plsc.VectorSubcoreMesh(core_axis_name="c", subcore_axis_name="s")

    @functools.partial(
        pl.kernel,
        out_type=jax.ShapeDtypeStruct((B, L), jnp.float32),
        mesh=mesh,
        scratch_types=[
            pltpu.VMEM((B,), jnp.int32),
            pltpu.VMEM((B, L), jnp.float32),
            pltpu.SemaphoreType.DMA,
        ],
    )
    def gather_kernel(table_hbm, idx_hbm, out_hbm, idx_v, rows_v, sem):
        first = (lax.axis_index("c") == 0) & (lax.axis_index("s") == 0)

        @pl.when(first)
        def _():
            pltpu.sync_copy(idx_hbm, idx_v)
            pltpu.async_copy(table_hbm.at[idx_v], rows_v, sem).wait()
            pltpu.sync_copy(rows_v, out_hbm)

    return gather_kernel(latents, idx)


def _trunk_body(x_ref, lat_ref, At_ref, Bt_ref, bias_ref, GM_ref, gb_ref,
                mut_ref, gamn_ref, WoutT_ref, bout_ref, out_ref):
    bf = jnp.bfloat16
    f32 = jnp.float32
    nn = (((1,), (0,)), ((), ()))
    H = At_ref.shape[1]

    lat = lat_ref[0].astype(bf)
    ll = [
        lax.dot_general(lat, Bt_ref[i], nn, preferred_element_type=f32)
        + bias_ref[i]
        for i in range(_NL + 1)
    ]

    x0 = x_ref[0, :, 0:1]
    x1 = x_ref[0, :, 1:2]
    xsq = x0 * x0 + x1 * x1
    xb = x_ref[0].astype(bf)

    def gabor(i):
        m0 = mut_ref[i, 0:1, :]
        m1 = mut_ref[i, 1:2, :]
        musq = m0 * m0 + m1 * m1
        sm = lax.dot_general(xb, GM_ref[i], nn, preferred_element_type=f32)
        e = jnp.exp2(((xsq + musq) - sm[:, H:]) * gamn_ref[i])
        return _fast_sin_phase(sm[:, :H] * _INV_PI + gb_ref[i]) * e

    z = gabor(0) * ll[0]
    for i in range(1, _NL + 1):
        zi = lax.dot_general(z.astype(bf), At_ref[i - 1],
                             (((1,), (0,)), ((), ())),
                             preferred_element_type=f32)
        z = (zi + ll[i]) * gabor(i)

    res = lax.dot_general(z.astype(bf), WoutT_ref[...],
                          (((1,), (0,)), ((), ())),
                          preferred_element_type=f32) + bout_ref[...]
    out_ref[...] = res[None]


def kernel(x, idx, latents, A0, A_rest, B_all, bias_all, gW, gb, mu, gamma,
           Wout, bout):
    del A0
    B, P, _ = x.shape
    H = A_rest.shape[1]
    L = B_all.shape[2]
    T = _TILE
    PT = P // T

    latent = _sc_gather(latents, idx)[:, None, :]

    At = A_rest.transpose(0, 2, 1).astype(jnp.bfloat16)
    Bt = B_all.transpose(0, 2, 1).astype(jnp.bfloat16)
    Wcat = gW.transpose(0, 2, 1).astype(jnp.bfloat16)
    mut = mu.transpose(0, 2, 1)
    Mcat = mut.astype(jnp.bfloat16) * jnp.bfloat16(2.0)
    GM = jnp.concatenate([Wcat, Mcat], axis=2)
    bias3 = bias_all[:, None, :]
    gb3 = gb[:, None, :] * jnp.float32(_INV_PI)
    gamn = gamma[:, None, :] * jnp.float32(-0.5 * 1.4426950408889634)
    WoutT = Wout.T.astype(jnp.bfloat16)
    bout2 = bout[None, :]

    full = lambda s: pl.BlockSpec(s, lambda b, p: (0,) * len(s))
    out = pl.pallas_call(
        _trunk_body,
        grid=(B, PT),
        in_specs=[
            pl.BlockSpec((1, T, 2), lambda b, p: (b, p, 0)),
            pl.BlockSpec((1, 1, L), lambda b, p: (b, 0, 0)),
            full((_NL, H, H)),
            full((_NL + 1, L, H)),
            full((_NL + 1, 1, H)),
            full((_NL + 1, 2, 2 * H)),
            full((_NL + 1, 1, H)),
            full((_NL + 1, 2, H)),
            full((_NL + 1, 1, H)),
            full((H, 1)),
            full((1, 1)),
        ],
        out_specs=pl.BlockSpec((1, T, 1), lambda b, p: (b, p, 0)),
        out_shape=jax.ShapeDtypeStruct((B, P, 1), jnp.float32),
        compiler_params=pltpu.CompilerParams(
            dimension_semantics=("parallel", "parallel"),
        ),
    )(x, latent, At, Bt, bias3, GM, gb3, mut, gamn, WoutT, bout2)
    return out
